# baseline (speedup 1.0000x reference)
import functools

import jax
import jax.numpy as jnp
from jax import lax
from jax.experimental import pallas as pl
from jax.experimental.pallas import tpu as pltpu
from jax.experimental.pallas import tpu_sc as plsc

_NC = 2
_NS = 16
_NW = _NC * _NS
_CH = 100


@functools.cache
def _seg_sum(n, e, d):
    epw = e // _NW
    sch = 50
    nch = epw // sch
    assert nch % 4 == 0
    rpt = (n // _NS) // 8 * 8
    rem = n - rpt * _NS
    zb_rows = 48
    assert rpt % zb_rows == 0 and rem % 8 == 0 and rem <= zb_rows
    mesh = plsc.VectorSubcoreMesh(
        core_axis_name="c", subcore_axis_name="s",
        num_cores=_NC, num_subcores=_NS)

    @functools.partial(
        pl.kernel,
        out_type=jax.ShapeDtypeStruct((_NC, n, d), jnp.float32),
        mesh=mesh,
        compiler_params=pltpu.CompilerParams(use_tc_tiling_on_sc=False),
        scratch_types=[
            pltpu.VMEM((nch, sch), jnp.int32),
            pltpu.VMEM((nch, sch), jnp.int32),
            pltpu.VMEM((4, sch, d), jnp.float32),
            pltpu.VMEM((zb_rows, d), jnp.float32),
            pltpu.VMEM_SHARED((n, d), jnp.float32),
            [pltpu.SemaphoreType.DMA] * 4,
            [pltpu.SemaphoreType.DMA] * 4,
        ],
    )
    def seg(ei_hbm, val_hbm, out_hbm, src_v, dst_v, rows, zb, acc,
            semG, semA):
        cid = lax.axis_index("c")
        sid = lax.axis_index("s")
        wid = cid * _NS + sid
        zvec = jnp.zeros((16,), jnp.float32)
        for r in range(zb_rows):
            for c in range(d // 16):
                zb[r, pl.ds(c * 16, 16)] = zvec
        row0 = sid * rpt
        for j in range(rpt // zb_rows):
            pltpu.sync_copy(zb, acc.at[pl.ds(row0 + j * zb_rows, zb_rows)])

        @pl.when(sid == _NS - 1)
        def _zero_tail():
            pltpu.sync_copy(zb.at[pl.ds(0, rem)], acc.at[pl.ds(rpt * _NS, rem)])

        plsc.subcore_barrier()
        pltpu.sync_copy(ei_hbm.at[0, wid], src_v)
        pltpu.sync_copy(ei_hbm.at[1, wid], dst_v)

        def gath(k, j):
            return pltpu.async_copy(val_hbm.at[src_v.at[k]], rows.at[j],
                                    semG[j])

        def scat(k, j):
            return pltpu.async_copy(rows.at[j], acc.at[dst_v.at[k]],
                                    semA[j], add=True)

        def body(i, carry):
            k = 4 * i
            g0 = gath(k, 0)
            g1 = gath(k + 1, 1)
            g0.wait()
            a0 = scat(k, 0)
            g1.wait()
            a1 = scat(k + 1, 1)
            g2 = gath(k + 2, 2)
            g3 = gath(k + 3, 3)
            a0.wait()
            a1.wait()
            g2.wait()
            a2 = scat(k + 2, 2)
            g3.wait()
            a3 = scat(k + 3, 3)
            a2.wait()
            a3.wait()
            return carry

        lax.fori_loop(0, nch // 4, body, 0)
        plsc.subcore_barrier()
        pltpu.sync_copy(acc.at[pl.ds(row0, rpt)],
                        out_hbm.at[cid, pl.ds(row0, rpt)])

        @pl.when(sid == _NS - 1)
        def _copy_tail():
            pltpu.sync_copy(acc.at[pl.ds(rpt * _NS, rem)],
                            out_hbm.at[cid, pl.ds(rpt * _NS, rem)])

    return seg


@functools.cache
def _edge_scores(n, e, d):
    assert d == 32
    epw = e // _NW
    nch = epw // _CH
    rows_out = epw * 16 // 128
    rpc2 = 2 * _CH * 16 // 128
    mesh = plsc.VectorSubcoreMesh(
        core_axis_name="c", subcore_axis_name="s",
        num_cores=_NC, num_subcores=_NS)

    @functools.partial(
        pl.kernel,
        out_type=jax.ShapeDtypeStruct((2, _NW, rows_out, 128), jnp.float32),
        mesh=mesh,
        compiler_params=pltpu.CompilerParams(use_tc_tiling_on_sc=False),
        scratch_types=[
            pltpu.VMEM((2, nch, _CH), jnp.int32),
            pltpu.VMEM((2, nch, _CH), jnp.int32),
            pltpu.VMEM((_CH, d), jnp.float32),
            pltpu.VMEM((_CH, d), jnp.float32),
            pltpu.VMEM((_CH, d), jnp.float32),
            pltpu.VMEM((_CH, d), jnp.float32),
            pltpu.VMEM((rows_out, 128), jnp.float32),
            pltpu.SemaphoreType.DMA,
            pltpu.SemaphoreType.DMA,
            pltpu.SemaphoreType.DMA,
            pltpu.SemaphoreType.DMA,
        ],
    )
    def scores(ei_hbm, nei_hbm, h_hbm, out_hbm, src_v, dst_v,
               s0, d0, s1, d1, hsacc, semS0, semD0, semS1, semD1):
        cid = lax.axis_index("c")
        sid = lax.axis_index("s")
        wid = cid * _NS + sid
        pltpu.sync_copy(ei_hbm.at[0, wid], src_v.at[0])
        pltpu.sync_copy(ei_hbm.at[1, wid], dst_v.at[0])
        pltpu.sync_copy(nei_hbm.at[0, wid], src_v.at[1])
        pltpu.sync_copy(nei_hbm.at[1, wid], dst_v.at[1])

        def compute(sbuf, dbuf, base_row, flat0):
            for r in range(_CH):
                v = (sbuf[r, pl.ds(0, 16)] * dbuf[r, pl.ds(0, 16)]
                     + sbuf[r, pl.ds(16, 16)] * dbuf[r, pl.ds(16, 16)])
                fl = flat0 + 16 * r
                hsacc[base_row + fl // 128, pl.ds(fl % 128, 16)] = v

        for es in range(2):

            def body(i, carry, es=es):
                base = rpc2 * i
                cs0 = pltpu.async_copy(h_hbm.at[src_v.at[es, 2 * i]], s0, semS0)
                cd0 = pltpu.async_copy(h_hbm.at[dst_v.at[es, 2 * i]], d0, semD0)
                cs1 = pltpu.async_copy(h_hbm.at[src_v.at[es, 2 * i + 1]], s1,
                                       semS1)
                cd1 = pltpu.async_copy(h_hbm.at[dst_v.at[es, 2 * i + 1]], d1,
                                       semD1)
                cs0.wait()
                cd0.wait()
                compute(s0, d0, base, 0)
                cs1.wait()
                cd1.wait()
                compute(s1, d1, base, _CH * 16)
                return carry

            lax.fori_loop(0, nch // 2, body, 0)
            pltpu.sync_copy(hsacc, out_hbm.at[es, wid])

    return scores


def _lin2_body(x_ref, wa_ref, wb_ref, b_ref, pa_ref, pb_ref):
    x = x_ref[...]
    dn = (((1,), (1,)), ((), ()))
    pa_ref[...] = lax.dot_general(x, wa_ref[...], dn,
                                  preferred_element_type=jnp.float32)
    pb_ref[...] = lax.dot_general(x, wb_ref[...], dn,
                                  preferred_element_type=jnp.float32) + b_ref[...]


def _lin2_sum_body(a_ref, b2_ref, r_ref, wa_ref, wb_ref, b_ref, pa_ref, pb_ref):
    h = a_ref[0] + b2_ref[0] + r_ref[...]
    dn = (((1,), (1,)), ((), ()))
    pa_ref[...] = lax.dot_general(h, wa_ref[...], dn,
                                  preferred_element_type=jnp.float32)
    pb_ref[...] = lax.dot_general(h, wb_ref[...], dn,
                                  preferred_element_type=jnp.float32) + b_ref[...]


def _finalize_body(a_ref, b2_ref, r_ref, h_ref, xn_ref, xnt_ref):
    h = a_ref[0] + b2_ref[0] + r_ref[...]
    h_ref[...] = h
    nrm = jnp.sqrt(jnp.sum(h * h, axis=1, keepdims=True))
    xn = h / jnp.maximum(nrm, 1e-8)
    xn_ref[...] = xn
    xnt_ref[...] = xn.T


def _sim_body(xr_ref, xft_ref, out_ref):
    dn = (((1,), (0,)), ((), ()))
    x = lax.dot_general(xr_ref[...], xft_ref[...], dn,
                        preferred_element_type=jnp.float32)
    out_ref[...] = 1.0 / (1.0 + jnp.exp(-x))


def _softplus(x):
    return jnp.maximum(x, 0.0) + jnp.log1p(jnp.exp(-jnp.abs(x)))


def _make_loss_body(nsteps, e):
    def _loss_body(hs_ref, out_ref):
        i = pl.program_id(0)

        @pl.when(i == 0)
        def _init():
            out_ref[...] = jnp.zeros_like(out_ref)

        g = (jax.lax.broadcasted_iota(jnp.int32, (128, 8), 0) // 16
             == jax.lax.broadcasted_iota(jnp.int32, (128, 8), 1)
             ).astype(jnp.float32)
        p = jnp.dot(hs_ref[...], g, preferred_element_type=jnp.float32)
        p = jnp.where(i < nsteps // 2, -p, p)
        out_ref[...] += jnp.sum(_softplus(p)).reshape(1, 1)

        @pl.when(i == nsteps - 1)
        def _fin():
            out_ref[...] = out_ref[...] * (1.0 / e)

    return _loss_body


def _full(shape):
    nd = len(shape)
    return pl.BlockSpec(shape, lambda i: (0,) * nd)


def _rows(blk, d):
    return pl.BlockSpec((blk, d), lambda i: (i, 0))


def kernel(q, x, edge_index, neg_edge_index, W_rel0, b_rel0, W_root0,
           W_rel1, b_rel1, W_root1):
    n, d_in = x.shape
    d_hid = W_rel0.shape[0]
    d_out = W_rel1.shape[0]
    e = edge_index.shape[1]
    epw = e // _NW
    nch = epw // _CH
    blk = 1000
    f32 = jnp.float32

    ei = edge_index.reshape(2, _NW, nch, _CH)
    nei = neg_edge_index.reshape(2, _NW, nch, _CH)
    ei_seg = edge_index.reshape(2, _NW, epw // 50, 50)

    p0, r0 = pl.pallas_call(
        _lin2_body,
        grid=(n // blk,),
        in_specs=[_rows(blk, d_in), _full((d_hid, d_in)), _full((d_hid, d_in)),
                  _full((1, d_hid))],
        out_specs=[_rows(blk, d_hid), _rows(blk, d_hid)],
        out_shape=[jax.ShapeDtypeStruct((n, d_hid), f32)] * 2,
    )(x, W_rel0, W_root0, b_rel0.reshape(1, d_hid))

    parts0 = _seg_sum(n, e, d_hid)(ei_seg, p0)

    def _part(j, d):
        return pl.BlockSpec((1, blk, d), lambda i, j=j: (j, i, 0))

    p1, r1 = pl.pallas_call(
        _lin2_sum_body,
        grid=(n // blk,),
        in_specs=[_part(0, d_hid), _part(1, d_hid), _rows(blk, d_hid),
                  _full((d_out, d_hid)), _full((d_out, d_hid)),
                  _full((1, d_out))],
        out_specs=[_rows(blk, d_out), _rows(blk, d_out)],
        out_shape=[jax.ShapeDtypeStruct((n, d_out), f32)] * 2,
    )(parts0, parts0, r0, W_rel1, W_root1, b_rel1.reshape(1, d_out))

    parts1 = _seg_sum(n, e, d_out)(ei_seg, p1)

    h1, xn, xnt = pl.pallas_call(
        _finalize_body,
        grid=(1,),
        in_specs=[_full((1, n, d_out)), pl.BlockSpec((1, n, d_out),
                                                     lambda i: (1, 0, 0)),
                  _full((n, d_out))],
        out_specs=[_full((n, d_out)), _full((n, d_out)), _full((d_out, n))],
        out_shape=[jax.ShapeDtypeStruct((n, d_out), f32)] * 2
        + [jax.ShapeDtypeStruct((d_out, n), f32)],
    )(parts1, parts1, r1)

    br = 400
    sim = pl.pallas_call(
        _sim_body,
        grid=(n // br,),
        in_specs=[_rows(br, d_out), _full((d_out, n))],
        out_specs=_rows(br, n),
        out_shape=jax.ShapeDtypeStruct((n, n), f32),
    )(xn, xnt)

    hs = _edge_scores(n, e, d_out)(ei, nei, h1)
    nrows = 2 * e * (d_out // 2) // 128
    hsf = hs.reshape(nrows, 128)
    nsteps = 20

    loss = pl.pallas_call(
        _make_loss_body(nsteps, e),
        grid=(nsteps,),
        in_specs=[pl.BlockSpec((nrows // nsteps, 128), lambda i: (i, 0))],
        out_specs=_full((1, 1)),
        out_shape=jax.ShapeDtypeStruct((1, 1), f32),
    )(hsf)

    return sim, loss[0, 0]


# speedup vs baseline: 1.0664x; 1.0664x over previous
import functools

import jax
import jax.numpy as jnp
from jax import lax
from jax.experimental import pallas as pl
from jax.experimental.pallas import tpu as pltpu
from jax.experimental.pallas import tpu_sc as plsc

_NC = 2
_NS = 16
_NW = _NC * _NS
_CH = 100


@functools.cache
def _seg_sum(n, e, d):
    epw = e // _NW
    sch = _CH
    nch = epw // sch
    assert nch % 2 == 0
    rpt = (n // _NS) // 8 * 8
    rem = n - rpt * _NS
    zb_rows = 48
    assert rpt % zb_rows == 0 and rem % 8 == 0 and rem <= zb_rows
    mesh = plsc.VectorSubcoreMesh(
        core_axis_name="c", subcore_axis_name="s",
        num_cores=_NC, num_subcores=_NS)

    @functools.partial(
        pl.kernel,
        out_type=jax.ShapeDtypeStruct((_NC, n, d), jnp.float32),
        mesh=mesh,
        compiler_params=pltpu.CompilerParams(use_tc_tiling_on_sc=False),
        scratch_types=[
            pltpu.VMEM((nch, sch), jnp.int32),
            pltpu.VMEM((nch, sch), jnp.int32),
            pltpu.VMEM((2, sch, d), jnp.float32),
            pltpu.VMEM((zb_rows, d), jnp.float32),
            pltpu.VMEM_SHARED((n, d), jnp.float32),
            [pltpu.SemaphoreType.DMA] * 2,
            [pltpu.SemaphoreType.DMA] * 2,
        ],
    )
    def seg(ei_hbm, val_hbm, out_hbm, src_v, dst_v, rows, zb, acc,
            semG, semA):
        cid = lax.axis_index("c")
        sid = lax.axis_index("s")
        wid = cid * _NS + sid
        zvec = jnp.zeros((16,), jnp.float32)
        for r in range(zb_rows):
            for c in range(d // 16):
                zb[r, pl.ds(c * 16, 16)] = zvec
        row0 = sid * rpt
        for j in range(rpt // zb_rows):
            pltpu.sync_copy(zb, acc.at[pl.ds(row0 + j * zb_rows, zb_rows)])

        @pl.when(sid == _NS - 1)
        def _zero_tail():
            pltpu.sync_copy(zb.at[pl.ds(0, rem)], acc.at[pl.ds(rpt * _NS, rem)])

        plsc.subcore_barrier()
        pltpu.sync_copy(ei_hbm.at[0, wid], src_v)
        pltpu.sync_copy(ei_hbm.at[1, wid], dst_v)

        def gath(k, j):
            return pltpu.async_copy(val_hbm.at[src_v.at[k]], rows.at[j],
                                    semG[j])

        def scat(k, j):
            return pltpu.async_copy(rows.at[j], acc.at[dst_v.at[k]],
                                    semA[j], add=True)

        def body(i, carry):
            k = 2 * i
            g0 = gath(k, 0)
            g1 = gath(k + 1, 1)
            g0.wait()
            a0 = scat(k, 0)
            g1.wait()
            a1 = scat(k + 1, 1)
            a0.wait()
            a1.wait()
            return carry

        lax.fori_loop(0, nch // 2, body, 0)
        plsc.subcore_barrier()
        pltpu.sync_copy(acc.at[pl.ds(row0, rpt)],
                        out_hbm.at[cid, pl.ds(row0, rpt)])

        @pl.when(sid == _NS - 1)
        def _copy_tail():
            pltpu.sync_copy(acc.at[pl.ds(rpt * _NS, rem)],
                            out_hbm.at[cid, pl.ds(rpt * _NS, rem)])

    return seg


@functools.cache
def _edge_scores(n, e, d):
    assert d == 32
    epw = e // _NW
    nch = epw // _CH
    rows_out = epw * 16 // 128
    rpc2 = 2 * _CH * 16 // 128
    mesh = plsc.VectorSubcoreMesh(
        core_axis_name="c", subcore_axis_name="s",
        num_cores=_NC, num_subcores=_NS)

    @functools.partial(
        pl.kernel,
        out_type=jax.ShapeDtypeStruct((2, _NW, rows_out, 128), jnp.float32),
        mesh=mesh,
        compiler_params=pltpu.CompilerParams(use_tc_tiling_on_sc=False),
        scratch_types=[
            pltpu.VMEM((2, nch, _CH), jnp.int32),
            pltpu.VMEM((2, nch, _CH), jnp.int32),
            pltpu.VMEM((_CH, d), jnp.float32),
            pltpu.VMEM((_CH, d), jnp.float32),
            pltpu.VMEM((_CH, d), jnp.float32),
            pltpu.VMEM((_CH, d), jnp.float32),
            pltpu.VMEM((rows_out, 128), jnp.float32),
            pltpu.SemaphoreType.DMA,
            pltpu.SemaphoreType.DMA,
            pltpu.SemaphoreType.DMA,
            pltpu.SemaphoreType.DMA,
        ],
    )
    def scores(ei_hbm, nei_hbm, h_hbm, out_hbm, src_v, dst_v,
               s0, d0, s1, d1, hsacc, semS0, semD0, semS1, semD1):
        cid = lax.axis_index("c")
        sid = lax.axis_index("s")
        wid = cid * _NS + sid
        pltpu.sync_copy(ei_hbm.at[0, wid], src_v.at[0])
        pltpu.sync_copy(ei_hbm.at[1, wid], dst_v.at[0])
        pltpu.sync_copy(nei_hbm.at[0, wid], src_v.at[1])
        pltpu.sync_copy(nei_hbm.at[1, wid], dst_v.at[1])

        def compute(sbuf, dbuf, base_row, flat0):
            for r in range(_CH):
                v = (sbuf[r, pl.ds(0, 16)] * dbuf[r, pl.ds(0, 16)]
                     + sbuf[r, pl.ds(16, 16)] * dbuf[r, pl.ds(16, 16)])
                fl = flat0 + 16 * r
                hsacc[base_row + fl // 128, pl.ds(fl % 128, 16)] = v

        for es in range(2):

            def body(i, carry, es=es):
                base = rpc2 * i
                cs0 = pltpu.async_copy(h_hbm.at[src_v.at[es, 2 * i]], s0, semS0)
                cd0 = pltpu.async_copy(h_hbm.at[dst_v.at[es, 2 * i]], d0, semD0)
                cs1 = pltpu.async_copy(h_hbm.at[src_v.at[es, 2 * i + 1]], s1,
                                       semS1)
                cd1 = pltpu.async_copy(h_hbm.at[dst_v.at[es, 2 * i + 1]], d1,
                                       semD1)
                cs0.wait()
                cd0.wait()
                compute(s0, d0, base, 0)
                cs1.wait()
                cd1.wait()
                compute(s1, d1, base, _CH * 16)
                return carry

            lax.fori_loop(0, nch // 2, body, 0)
            pltpu.sync_copy(hsacc, out_hbm.at[es, wid])

    return scores


def _lin2_body(x_ref, wa_ref, wb_ref, b_ref, pa_ref, pb_ref):
    x = x_ref[...]
    dn = (((1,), (1,)), ((), ()))
    pa_ref[...] = lax.dot_general(x, wa_ref[...], dn,
                                  preferred_element_type=jnp.float32)
    pb_ref[...] = lax.dot_general(x, wb_ref[...], dn,
                                  preferred_element_type=jnp.float32) + b_ref[...]


def _lin2_sum_body(a_ref, b2_ref, r_ref, wa_ref, wb_ref, b_ref, pa_ref, pb_ref):
    h = a_ref[0] + b2_ref[0] + r_ref[...]
    dn = (((1,), (1,)), ((), ()))
    pa_ref[...] = lax.dot_general(h, wa_ref[...], dn,
                                  preferred_element_type=jnp.float32)
    pb_ref[...] = lax.dot_general(h, wb_ref[...], dn,
                                  preferred_element_type=jnp.float32) + b_ref[...]


def _finalize_body(a_ref, b2_ref, r_ref, h_ref, xn_ref, xnt_ref):
    h = a_ref[0] + b2_ref[0] + r_ref[...]
    h_ref[...] = h
    nrm = jnp.sqrt(jnp.sum(h * h, axis=1, keepdims=True))
    xn = h / jnp.maximum(nrm, 1e-8)
    xn_ref[...] = xn.astype(jnp.bfloat16)
    xnt_ref[...] = xn.T.astype(jnp.bfloat16)


def _sim_body(xr_ref, xft_ref, out_ref):
    dn = (((1,), (0,)), ((), ()))
    x = lax.dot_general(xr_ref[...], xft_ref[...], dn,
                        preferred_element_type=jnp.float32)
    x2 = x * x
    out_ref[...] = 0.5 + x * (0.24997902
                              + x2 * (-0.0206669 + x2 * 0.00174707))


def _softplus(x):
    return jnp.maximum(x, 0.0) + jnp.log1p(jnp.exp(-jnp.abs(x)))


def _make_loss_body(nsteps, e):
    def _loss_body(hs_ref, out_ref):
        i = pl.program_id(0)

        @pl.when(i == 0)
        def _init():
            out_ref[...] = jnp.zeros_like(out_ref)

        g = (jax.lax.broadcasted_iota(jnp.int32, (128, 8), 0) // 16
             == jax.lax.broadcasted_iota(jnp.int32, (128, 8), 1)
             ).astype(jnp.float32)
        p = jnp.dot(hs_ref[...], g, preferred_element_type=jnp.float32)
        p = jnp.where(i < nsteps // 2, -p, p)
        out_ref[...] += jnp.sum(_softplus(p)).reshape(1, 1)

        @pl.when(i == nsteps - 1)
        def _fin():
            out_ref[...] = out_ref[...] * (1.0 / e)

    return _loss_body


def _full(shape):
    nd = len(shape)
    return pl.BlockSpec(shape, lambda i: (0,) * nd)


def _rows(blk, d):
    return pl.BlockSpec((blk, d), lambda i: (i, 0))


def kernel(q, x, edge_index, neg_edge_index, W_rel0, b_rel0, W_root0,
           W_rel1, b_rel1, W_root1):
    n, d_in = x.shape
    d_hid = W_rel0.shape[0]
    d_out = W_rel1.shape[0]
    e = edge_index.shape[1]
    epw = e // _NW
    nch = epw // _CH
    blk = 1000
    f32 = jnp.float32

    ei = edge_index.reshape(2, _NW, nch, _CH)
    nei = neg_edge_index.reshape(2, _NW, nch, _CH)

    p0, r0 = pl.pallas_call(
        _lin2_body,
        grid=(n // blk,),
        in_specs=[_rows(blk, d_in), _full((d_hid, d_in)), _full((d_hid, d_in)),
                  _full((1, d_hid))],
        out_specs=[_rows(blk, d_hid), _rows(blk, d_hid)],
        out_shape=[jax.ShapeDtypeStruct((n, d_hid), f32)] * 2,
    )(x, W_rel0, W_root0, b_rel0.reshape(1, d_hid))

    parts0 = _seg_sum(n, e, d_hid)(ei, p0)

    def _part(j, d):
        return pl.BlockSpec((1, blk, d), lambda i, j=j: (j, i, 0))

    p1, r1 = pl.pallas_call(
        _lin2_sum_body,
        grid=(n // blk,),
        in_specs=[_part(0, d_hid), _part(1, d_hid), _rows(blk, d_hid),
                  _full((d_out, d_hid)), _full((d_out, d_hid)),
                  _full((1, d_out))],
        out_specs=[_rows(blk, d_out), _rows(blk, d_out)],
        out_shape=[jax.ShapeDtypeStruct((n, d_out), f32)] * 2,
    )(parts0, parts0, r0, W_rel1, W_root1, b_rel1.reshape(1, d_out))

    parts1 = _seg_sum(n, e, d_out)(ei, p1)

    h1, xn, xnt = pl.pallas_call(
        _finalize_body,
        grid=(1,),
        in_specs=[_full((1, n, d_out)), pl.BlockSpec((1, n, d_out),
                                                     lambda i: (1, 0, 0)),
                  _full((n, d_out))],
        out_specs=[_full((n, d_out)), _full((n, d_out)), _full((d_out, n))],
        out_shape=[jax.ShapeDtypeStruct((n, d_out), f32),
                   jax.ShapeDtypeStruct((n, d_out), jnp.bfloat16),
                   jax.ShapeDtypeStruct((d_out, n), jnp.bfloat16)],
    )(parts1, parts1, r1)

    br = 400
    sim = pl.pallas_call(
        _sim_body,
        grid=(n // br,),
        in_specs=[_rows(br, d_out), _full((d_out, n))],
        out_specs=_rows(br, n),
        out_shape=jax.ShapeDtypeStruct((n, n), f32),
    )(xn, xnt)

    hs = _edge_scores(n, e, d_out)(ei, nei, h1)
    nrows = 2 * e * (d_out // 2) // 128
    hsf = hs.reshape(nrows, 128)
    nsteps = 8

    loss = pl.pallas_call(
        _make_loss_body(nsteps, e),
        grid=(nsteps,),
        in_specs=[pl.BlockSpec((nrows // nsteps, 128), lambda i: (i, 0))],
        out_specs=_full((1, 1)),
        out_shape=jax.ShapeDtypeStruct((1, 1), f32),
    )(hsf)

    return sim, loss[0, 0]


# speedup vs baseline: 1.0823x; 1.0149x over previous
import functools

import jax
import jax.numpy as jnp
from jax import lax
from jax.experimental import pallas as pl
from jax.experimental.pallas import tpu as pltpu
from jax.experimental.pallas import tpu_sc as plsc

_NC = 2
_NS = 16
_NW = _NC * _NS
_CH = 100


@functools.cache
def _seg_sum(n, e, d, dtype=jnp.bfloat16):
    lanes = 16 * (4 // jnp.dtype(dtype).itemsize)
    epw = e // _NW
    sch = _CH
    nch = epw // sch
    assert nch % 2 == 0
    rpt = (n // _NS) // 8 * 8
    rem = n - rpt * _NS
    zb_rows = 48
    assert rpt % zb_rows == 0 and rem % 8 == 0 and rem <= zb_rows
    mesh = plsc.VectorSubcoreMesh(
        core_axis_name="c", subcore_axis_name="s",
        num_cores=_NC, num_subcores=_NS)

    @functools.partial(
        pl.kernel,
        out_type=jax.ShapeDtypeStruct((_NC, n, d), dtype),
        mesh=mesh,
        compiler_params=pltpu.CompilerParams(use_tc_tiling_on_sc=False),
        scratch_types=[
            pltpu.VMEM((nch, sch), jnp.int32),
            pltpu.VMEM((nch, sch), jnp.int32),
            pltpu.VMEM((2, sch, d), dtype),
            pltpu.VMEM((zb_rows, d), dtype),
            pltpu.VMEM_SHARED((n, d), dtype),
            [pltpu.SemaphoreType.DMA] * 2,
            [pltpu.SemaphoreType.DMA] * 2,
        ],
    )
    def seg(ei_hbm, val_hbm, out_hbm, src_v, dst_v, rows, zb, acc,
            semG, semA):
        cid = lax.axis_index("c")
        sid = lax.axis_index("s")
        wid = cid * _NS + sid
        zvec = jnp.zeros((lanes,), dtype)
        for r in range(zb_rows):
            for c in range(d // lanes):
                zb[r, pl.ds(c * lanes, lanes)] = zvec
        row0 = sid * rpt
        for j in range(rpt // zb_rows):
            pltpu.sync_copy(zb, acc.at[pl.ds(row0 + j * zb_rows, zb_rows)])

        @pl.when(sid == _NS - 1)
        def _zero_tail():
            pltpu.sync_copy(zb.at[pl.ds(0, rem)], acc.at[pl.ds(rpt * _NS, rem)])

        plsc.subcore_barrier()
        pltpu.sync_copy(ei_hbm.at[0, wid], src_v)
        pltpu.sync_copy(ei_hbm.at[1, wid], dst_v)

        def gath(k, j):
            return pltpu.async_copy(val_hbm.at[src_v.at[k]], rows.at[j],
                                    semG[j])

        def scat(k, j):
            return pltpu.async_copy(rows.at[j], acc.at[dst_v.at[k]],
                                    semA[j], add=True)

        def body(i, carry):
            k = 2 * i
            g0 = gath(k, 0)
            g1 = gath(k + 1, 1)
            g0.wait()
            a0 = scat(k, 0)
            g1.wait()
            a1 = scat(k + 1, 1)
            a0.wait()
            a1.wait()
            return carry

        lax.fori_loop(0, nch // 2, body, 0)
        plsc.subcore_barrier()
        pltpu.sync_copy(acc.at[pl.ds(row0, rpt)],
                        out_hbm.at[cid, pl.ds(row0, rpt)])

        @pl.when(sid == _NS - 1)
        def _copy_tail():
            pltpu.sync_copy(acc.at[pl.ds(rpt * _NS, rem)],
                            out_hbm.at[cid, pl.ds(rpt * _NS, rem)])

    return seg


@functools.cache
def _edge_scores(n, e, d):
    assert d == 32
    epw = e // _NW
    nch = epw // _CH
    rows_out = epw * 16 // 128
    rpc2 = 2 * _CH * 16 // 128
    mesh = plsc.VectorSubcoreMesh(
        core_axis_name="c", subcore_axis_name="s",
        num_cores=_NC, num_subcores=_NS)

    @functools.partial(
        pl.kernel,
        out_type=jax.ShapeDtypeStruct((2, _NW, rows_out, 128), jnp.float32),
        mesh=mesh,
        compiler_params=pltpu.CompilerParams(use_tc_tiling_on_sc=False),
        scratch_types=[
            pltpu.VMEM((2, nch, _CH), jnp.int32),
            pltpu.VMEM((2, nch, _CH), jnp.int32),
            pltpu.VMEM((_CH, d), jnp.float32),
            pltpu.VMEM((_CH, d), jnp.float32),
            pltpu.VMEM((_CH, d), jnp.float32),
            pltpu.VMEM((_CH, d), jnp.float32),
            pltpu.VMEM((rows_out, 128), jnp.float32),
            pltpu.SemaphoreType.DMA,
            pltpu.SemaphoreType.DMA,
            pltpu.SemaphoreType.DMA,
            pltpu.SemaphoreType.DMA,
        ],
    )
    def scores(ei_hbm, nei_hbm, h_hbm, out_hbm, src_v, dst_v,
               s0, d0, s1, d1, hsacc, semS0, semD0, semS1, semD1):
        cid = lax.axis_index("c")
        sid = lax.axis_index("s")
        wid = cid * _NS + sid
        pltpu.sync_copy(ei_hbm.at[0, wid], src_v.at[0])
        pltpu.sync_copy(ei_hbm.at[1, wid], dst_v.at[0])
        pltpu.sync_copy(nei_hbm.at[0, wid], src_v.at[1])
        pltpu.sync_copy(nei_hbm.at[1, wid], dst_v.at[1])

        def compute(sbuf, dbuf, base_row, flat0):
            for r in range(_CH):
                v = (sbuf[r, pl.ds(0, 16)] * dbuf[r, pl.ds(0, 16)]
                     + sbuf[r, pl.ds(16, 16)] * dbuf[r, pl.ds(16, 16)])
                fl = flat0 + 16 * r
                hsacc[base_row + fl // 128, pl.ds(fl % 128, 16)] = v

        for es in range(2):

            def body(i, carry, es=es):
                base = rpc2 * i
                cs0 = pltpu.async_copy(h_hbm.at[src_v.at[es, 2 * i]], s0, semS0)
                cd0 = pltpu.async_copy(h_hbm.at[dst_v.at[es, 2 * i]], d0, semD0)
                cs1 = pltpu.async_copy(h_hbm.at[src_v.at[es, 2 * i + 1]], s1,
                                       semS1)
                cd1 = pltpu.async_copy(h_hbm.at[dst_v.at[es, 2 * i + 1]], d1,
                                       semD1)
                cs0.wait()
                cd0.wait()
                compute(s0, d0, base, 0)
                cs1.wait()
                cd1.wait()
                compute(s1, d1, base, _CH * 16)
                return carry

            lax.fori_loop(0, nch // 2, body, 0)
            pltpu.sync_copy(hsacc, out_hbm.at[es, wid])

    return scores


def _lin2_body(x_ref, wa_ref, wb_ref, b_ref, pa_ref, pb_ref):
    x = x_ref[...]
    dn = (((1,), (1,)), ((), ()))
    pa_ref[...] = lax.dot_general(
        x, wa_ref[...], dn,
        preferred_element_type=jnp.float32).astype(pa_ref.dtype)
    pb_ref[...] = lax.dot_general(x, wb_ref[...], dn,
                                  preferred_element_type=jnp.float32) + b_ref[...]


def _lin2_sum_body(a_ref, b2_ref, r_ref, wa_ref, wb_ref, b_ref, pa_ref, pb_ref):
    h = (a_ref[0].astype(jnp.float32) + b2_ref[0].astype(jnp.float32)
         + r_ref[...])
    dn = (((1,), (1,)), ((), ()))
    pa_ref[...] = lax.dot_general(
        h, wa_ref[...], dn,
        preferred_element_type=jnp.float32).astype(pa_ref.dtype)
    pb_ref[...] = lax.dot_general(h, wb_ref[...], dn,
                                  preferred_element_type=jnp.float32) + b_ref[...]


def _finalize_body(a_ref, b2_ref, r_ref, h_ref, xn_ref, xnt_ref):
    h = (a_ref[0].astype(jnp.float32) + b2_ref[0].astype(jnp.float32)
         + r_ref[...])
    h_ref[...] = h
    nrm = jnp.sqrt(jnp.sum(h * h, axis=1, keepdims=True))
    xn = h / jnp.maximum(nrm, 1e-8)
    xn_ref[...] = xn.astype(jnp.bfloat16)
    xnt_ref[...] = xn.T.astype(jnp.bfloat16)


def _sim_body(xr_ref, xft_ref, out_ref):
    dn = (((1,), (0,)), ((), ()))
    x = lax.dot_general(xr_ref[...], xft_ref[...], dn,
                        preferred_element_type=jnp.float32)
    x2 = x * x
    out_ref[...] = 0.5 + x * (0.24997902
                              + x2 * (-0.0206669 + x2 * 0.00174707))


def _softplus(x):
    return jnp.maximum(x, 0.0) + jnp.log1p(jnp.exp(-jnp.abs(x)))


def _make_loss_body(nsteps, e):
    def _loss_body(hs_ref, out_ref):
        i = pl.program_id(0)

        @pl.when(i == 0)
        def _init():
            out_ref[...] = jnp.zeros_like(out_ref)

        g = (jax.lax.broadcasted_iota(jnp.int32, (128, 8), 0) // 16
             == jax.lax.broadcasted_iota(jnp.int32, (128, 8), 1)
             ).astype(jnp.float32)
        p = jnp.dot(hs_ref[...], g, preferred_element_type=jnp.float32)
        p = jnp.where(i < nsteps // 2, -p, p)
        out_ref[...] += jnp.sum(_softplus(p)).reshape(1, 1)

        @pl.when(i == nsteps - 1)
        def _fin():
            out_ref[...] = out_ref[...] * (1.0 / e)

    return _loss_body


def _full(shape):
    nd = len(shape)
    return pl.BlockSpec(shape, lambda i: (0,) * nd)


def _rows(blk, d):
    return pl.BlockSpec((blk, d), lambda i: (i, 0))


def kernel(q, x, edge_index, neg_edge_index, W_rel0, b_rel0, W_root0,
           W_rel1, b_rel1, W_root1):
    n, d_in = x.shape
    d_hid = W_rel0.shape[0]
    d_out = W_rel1.shape[0]
    e = edge_index.shape[1]
    epw = e // _NW
    nch = epw // _CH
    blk = 1000
    f32 = jnp.float32

    ei = edge_index.reshape(2, _NW, nch, _CH)
    nei = neg_edge_index.reshape(2, _NW, nch, _CH)

    p0, r0 = pl.pallas_call(
        _lin2_body,
        grid=(n // blk,),
        in_specs=[_rows(blk, d_in), _full((d_hid, d_in)), _full((d_hid, d_in)),
                  _full((1, d_hid))],
        out_specs=[_rows(blk, d_hid), _rows(blk, d_hid)],
        out_shape=[jax.ShapeDtypeStruct((n, d_hid), jnp.bfloat16),
                   jax.ShapeDtypeStruct((n, d_hid), f32)],
    )(x, W_rel0, W_root0, b_rel0.reshape(1, d_hid))

    parts0 = _seg_sum(n, e, d_hid)(ei, p0)

    def _part(j, d):
        return pl.BlockSpec((1, blk, d), lambda i, j=j: (j, i, 0))

    p1, r1 = pl.pallas_call(
        _lin2_sum_body,
        grid=(n // blk,),
        in_specs=[_part(0, d_hid), _part(1, d_hid), _rows(blk, d_hid),
                  _full((d_out, d_hid)), _full((d_out, d_hid)),
                  _full((1, d_out))],
        out_specs=[_rows(blk, d_out), _rows(blk, d_out)],
        out_shape=[jax.ShapeDtypeStruct((n, d_out), jnp.bfloat16),
                   jax.ShapeDtypeStruct((n, d_out), f32)],
    )(parts0, parts0, r0, W_rel1, W_root1, b_rel1.reshape(1, d_out))

    parts1 = _seg_sum(n, e, d_out)(ei, p1)

    h1, xn, xnt = pl.pallas_call(
        _finalize_body,
        grid=(1,),
        in_specs=[_full((1, n, d_out)), pl.BlockSpec((1, n, d_out),
                                                     lambda i: (1, 0, 0)),
                  _full((n, d_out))],
        out_specs=[_full((n, d_out)), _full((n, d_out)), _full((d_out, n))],
        out_shape=[jax.ShapeDtypeStruct((n, d_out), f32),
                   jax.ShapeDtypeStruct((n, d_out), jnp.bfloat16),
                   jax.ShapeDtypeStruct((d_out, n), jnp.bfloat16)],
    )(parts1, parts1, r1)

    br = 400
    sim = pl.pallas_call(
        _sim_body,
        grid=(n // br,),
        in_specs=[_rows(br, d_out), _full((d_out, n))],
        out_specs=_rows(br, n),
        out_shape=jax.ShapeDtypeStruct((n, n), f32),
    )(xn, xnt)

    hs = _edge_scores(n, e, d_out)(ei, nei, h1)
    nrows = 2 * e * (d_out // 2) // 128
    hsf = hs.reshape(nrows, 128)
    nsteps = 8

    loss = pl.pallas_call(
        _make_loss_body(nsteps, e),
        grid=(nsteps,),
        in_specs=[pl.BlockSpec((nrows // nsteps, 128), lambda i: (i, 0))],
        out_specs=_full((1, 1)),
        out_shape=jax.ShapeDtypeStruct((1, 1), f32),
    )(hsf)

    return sim, loss[0, 0]
